# baseline (speedup 1.0000x reference)
import numpy as np
import jax
import jax.numpy as jnp
from jax.experimental import pallas as pl
from jax.experimental.pallas import tpu as pltpu

POINTNUMS = 32
HS = 2016
IN = 128
PROJ = 504
CLIP = 2.0
G_RATIO = 150.0 / CLIP

HSP = 2048


def _ramp_thresholds(v):
    dv = np.diff(v)
    r = np.round(dv, 3)
    cell = r / r.min()
    g = 150.0 / cell.max() * cell
    dva = g * (dv.max() / 150.0)
    V = np.zeros(len(v), dtype=np.float64)
    mid = len(dv) // 2
    V[0] = -dva[:mid].sum()
    V[1:] = V[0] + np.cumsum(dva)
    return V[:POINTNUMS - 1].astype(np.float32)


_t_s = np.arange(1, POINTNUMS + 2) / float(POINTNUMS + 2)
SIG_THR_LIST = [float(t) for t in _ramp_thresholds(np.log(_t_s / (1.0 - _t_s)))]
_PT = POINTNUMS // 2
_t_t = (np.arange(2 * _PT + 1) - _PT) / float(_PT + 1)
TANH_THR_LIST = [float(t) for t in
                 _ramp_thresholds(0.5 * np.log((1.0 + _t_t) / (1.0 - _t_t)))]
N_THR = len(SIG_THR_LIST)

def _prep_body(raw_ref, delg_ref, out_ref):
    out_ref[...] = delg_ref[...] / G_RATIO + jnp.clip(raw_ref[...], -CLIP, CLIP)


def _count_chain(v, thr):
    def tsel(masks_weights, offset):
        if not masks_weights:
            return thr[offset]
        (m, w) = masks_weights[0]
        rest = masks_weights[1:]
        return jnp.where(m, tsel(rest, offset + w), tsel(rest, offset))

    m16 = v >= thr[15]
    m8 = v >= tsel([(m16, 16)], 7)
    m4 = v >= tsel([(m16, 16), (m8, 8)], 3)
    m2 = v >= tsel([(m16, 16), (m8, 8), (m4, 4)], 1)
    m1 = v >= tsel([(m16, 16), (m8, 8), (m4, 4), (m2, 2)], 0)
    cnt = jnp.where(m16, 16.0, 0.0)
    cnt = jnp.where(m8, cnt + 8.0, cnt)
    cnt = jnp.where(m4, cnt + 4.0, cnt)
    cnt = jnp.where(m2, cnt + 2.0, cnt)
    cnt = jnp.where(m1, cnt + 1.0, cnt)
    return cnt


def _lstm_body(x_ref, wx_ref, wh_ref, b_ref, pw_ref, pb_ref,
               hid_ref, h_ref, c_ref,
               h_scr, c_scr, ha_scr):
    t = pl.program_id(1)
    nt = pl.num_programs(1)

    @pl.when(t == 0)
    def _():
        h_scr[...] = jnp.zeros_like(h_scr)
        c_scr[...] = jnp.zeros_like(c_scr)

    xb = x_ref[0].astype(jnp.bfloat16)
    hb = h_scr[...].astype(jnp.bfloat16)
    gates = (jnp.dot(xb, wx_ref[...], preferred_element_type=jnp.float32)
             + jnp.dot(hb, wh_ref[...], preferred_element_type=jnp.float32)
             + b_ref[...])

    i_t = _count_chain(gates[:, 0:HSP], SIG_THR_LIST) * (1.0 / (POINTNUMS + 2))
    f_t = _count_chain(gates[:, HSP:2 * HSP], SIG_THR_LIST) * (1.0 / (POINTNUMS + 2))
    g_t = _count_chain(gates[:, 2 * HSP:3 * HSP], TANH_THR_LIST) * (2.0 / (POINTNUMS + 2)) - 1.0
    o_t = _count_chain(gates[:, 3 * HSP:4 * HSP], SIG_THR_LIST) * (1.0 / (POINTNUMS + 2))

    c = f_t * c_scr[...] + i_t * g_t
    c_scr[...] = c
    ot = (o_t * jnp.tanh(c)).astype(jnp.bfloat16)
    KH = HSP // 2
    ha_scr[...] = jnp.dot(ot[:, :KH], pw_ref[:KH, :],
                          preferred_element_type=jnp.float32)
    h = (ha_scr[...] + jnp.dot(ot[:, KH:], pw_ref[KH:, :],
                               preferred_element_type=jnp.float32)) + pb_ref[...]
    h_scr[...] = h

    hid_ref[...] = h.reshape(h.shape[0], 1, 1, PROJ)

    @pl.when(t == nt - 1)
    def _():
        h_ref[...] = h
        c_ref[...] = c[:, :HS]


def kernel(x, W, U, bias, proj_w, proj_b, delg):
    B, T, _ = x.shape
    BB = 256
    nb = B // BB

    raw = jnp.concatenate([W, U, bias[None, :]], axis=0)
    rawp = jnp.pad(raw, ((0, 7), (0, 0)))
    delgp = jnp.pad(delg, ((0, 7), (0, 0)))
    mu = pl.pallas_call(
        _prep_body,
        grid=(5,),
        in_specs=[pl.BlockSpec((128, 4 * HS), lambda i: (i, 0)),
                  pl.BlockSpec((128, 4 * HS), lambda i: (i, 0))],
        out_specs=pl.BlockSpec((128, 4 * HS), lambda i: (i, 0)),
        out_shape=jax.ShapeDtypeStruct((640, 4 * HS), jnp.float32),
        name="weight_prep",
    )(rawp, delgp)

    mup = jnp.pad(mu.reshape(640, 4, HS), ((0, 0), (0, 0), (0, HSP - HS)))
    mup = mup.reshape(640, 4 * HSP)
    w_x = mup[:IN].astype(jnp.bfloat16)
    w_h = mup[IN:IN + PROJ].astype(jnp.bfloat16)
    b_row = mup[IN + PROJ:IN + PROJ + 1]

    pw = jnp.pad(proj_w.T, ((0, HSP - HS), (0, 0))).astype(jnp.bfloat16)
    pb = proj_b[None, :]

    xt = jnp.swapaxes(x, 0, 1)

    hid, h_fin, c_fin = pl.pallas_call(
        _lstm_body,
        grid=(nb, T),
        in_specs=[
            pl.BlockSpec((1, BB, IN), lambda b, t: (t, b, 0)),
            pl.BlockSpec((IN, 4 * HSP), lambda b, t: (0, 0)),
            pl.BlockSpec((PROJ, 4 * HSP), lambda b, t: (0, 0)),
            pl.BlockSpec((1, 4 * HSP), lambda b, t: (0, 0)),
            pl.BlockSpec((HSP, PROJ), lambda b, t: (0, 0)),
            pl.BlockSpec((1, PROJ), lambda b, t: (0, 0)),
        ],
        out_specs=[
            pl.BlockSpec((BB, 1, 1, PROJ), lambda b, t: (b, t, 0, 0)),
            pl.BlockSpec((BB, PROJ), lambda b, t: (b, 0)),
            pl.BlockSpec((BB, HS), lambda b, t: (b, 0)),
        ],
        out_shape=[
            jax.ShapeDtypeStruct((B, T, 1, PROJ), jnp.float32),
            jax.ShapeDtypeStruct((B, PROJ), jnp.float32),
            jax.ShapeDtypeStruct((B, HS), jnp.float32),
        ],
        scratch_shapes=[
            pltpu.VMEM((BB, PROJ), jnp.float32),
            pltpu.VMEM((BB, HSP), jnp.float32),
            pltpu.VMEM((BB, PROJ), jnp.float32),
        ],
        compiler_params=pltpu.CompilerParams(
            dimension_semantics=("parallel", "arbitrary"),
            vmem_limit_bytes=56 * 1024 * 1024,
        ),
        name="lstm_adc",
    )(xt, w_x, w_h, b_row, pw, pb)

    return hid.reshape(B, T, PROJ), h_fin, c_fin


# speedup vs baseline: 1.0202x; 1.0202x over previous
import numpy as np
import jax
import jax.numpy as jnp
from jax.experimental import pallas as pl
from jax.experimental.pallas import tpu as pltpu

POINTNUMS = 32
HS = 2016
IN = 128
PROJ = 504
CLIP = 2.0
G_RATIO = 150.0 / CLIP

HSP = 2048


def _ramp_thresholds(v):
    dv = np.diff(v)
    r = np.round(dv, 3)
    cell = r / r.min()
    g = 150.0 / cell.max() * cell
    dva = g * (dv.max() / 150.0)
    V = np.zeros(len(v), dtype=np.float64)
    mid = len(dv) // 2
    V[0] = -dva[:mid].sum()
    V[1:] = V[0] + np.cumsum(dva)
    return V[:POINTNUMS - 1].astype(np.float32)


_t_s = np.arange(1, POINTNUMS + 2) / float(POINTNUMS + 2)
SIG_THR_LIST = [float(t) for t in _ramp_thresholds(np.log(_t_s / (1.0 - _t_s)))]
_PT = POINTNUMS // 2
_t_t = (np.arange(2 * _PT + 1) - _PT) / float(_PT + 1)
TANH_THR_LIST = [float(t) for t in
                 _ramp_thresholds(0.5 * np.log((1.0 + _t_t) / (1.0 - _t_t)))]
N_THR = len(SIG_THR_LIST)

def _prep_body(raw_ref, delg_ref, out_ref):
    out_ref[...] = delg_ref[...] / G_RATIO + jnp.clip(raw_ref[...], -CLIP, CLIP)


def _count_chain(v, thr):
    def tsel(masks_weights, offset):
        if not masks_weights:
            return thr[offset]
        (m, w) = masks_weights[0]
        rest = masks_weights[1:]
        return jnp.where(m, tsel(rest, offset + w), tsel(rest, offset))

    m16 = v >= thr[15]
    m8 = v >= tsel([(m16, 16)], 7)
    m4 = v >= tsel([(m16, 16), (m8, 8)], 3)
    m2 = v >= tsel([(m16, 16), (m8, 8), (m4, 4)], 1)
    m1 = v >= tsel([(m16, 16), (m8, 8), (m4, 4), (m2, 2)], 0)
    cnt = jnp.where(m16, 16.0, 0.0)
    cnt = jnp.where(m8, cnt + 8.0, cnt)
    cnt = jnp.where(m4, cnt + 4.0, cnt)
    cnt = jnp.where(m2, cnt + 2.0, cnt)
    cnt = jnp.where(m1, cnt + 1.0, cnt)
    return cnt


def _lstm_body(x_ref, wx_ref, wh_ref, b_ref, pw_ref, pb_ref,
               hid_ref, h_ref, c_ref,
               h_scr, c_scr, ha_scr):
    t = pl.program_id(1)
    nt = pl.num_programs(1)

    @pl.when(t == 0)
    def _():
        h_scr[...] = jnp.zeros_like(h_scr)
        c_scr[...] = jnp.zeros_like(c_scr)

    xb = x_ref[0].astype(jnp.bfloat16)
    hb = h_scr[...].astype(jnp.bfloat16)
    acts = []
    for k, thr in enumerate((SIG_THR_LIST, SIG_THR_LIST,
                             TANH_THR_LIST, SIG_THR_LIST)):
        sl = slice(k * HSP, (k + 1) * HSP)
        g_k = (jnp.dot(xb, wx_ref[:, sl], preferred_element_type=jnp.float32)
               + jnp.dot(hb, wh_ref[:, sl], preferred_element_type=jnp.float32)
               + b_ref[:, sl])
        acts.append(_count_chain(g_k, thr))

    i_t = acts[0] * (1.0 / (POINTNUMS + 2))
    f_t = acts[1] * (1.0 / (POINTNUMS + 2))
    g_t = acts[2] * (2.0 / (POINTNUMS + 2)) - 1.0
    o_t = acts[3] * (1.0 / (POINTNUMS + 2))

    c = f_t * c_scr[...] + i_t * g_t
    c_scr[...] = c
    ot = (o_t * jnp.tanh(c)).astype(jnp.bfloat16)
    KH = HSP // 2
    ha_scr[...] = jnp.dot(ot[:, :KH], pw_ref[:KH, :],
                          preferred_element_type=jnp.float32)
    h = (ha_scr[...] + jnp.dot(ot[:, KH:], pw_ref[KH:, :],
                               preferred_element_type=jnp.float32)) + pb_ref[...]
    h_scr[...] = h

    hid_ref[...] = h.reshape(h.shape[0], 1, 1, PROJ)

    @pl.when(t == nt - 1)
    def _():
        h_ref[...] = h
        c_ref[...] = c[:, :HS]


def kernel(x, W, U, bias, proj_w, proj_b, delg):
    B, T, _ = x.shape
    BB = 256
    nb = B // BB

    raw = jnp.concatenate([W, U, bias[None, :]], axis=0)
    rawp = jnp.pad(raw, ((0, 7), (0, 0)))
    delgp = jnp.pad(delg, ((0, 7), (0, 0)))
    mu = pl.pallas_call(
        _prep_body,
        grid=(5,),
        in_specs=[pl.BlockSpec((128, 4 * HS), lambda i: (i, 0)),
                  pl.BlockSpec((128, 4 * HS), lambda i: (i, 0))],
        out_specs=pl.BlockSpec((128, 4 * HS), lambda i: (i, 0)),
        out_shape=jax.ShapeDtypeStruct((640, 4 * HS), jnp.float32),
        name="weight_prep",
    )(rawp, delgp)

    mup = jnp.pad(mu.reshape(640, 4, HS), ((0, 0), (0, 0), (0, HSP - HS)))
    mup = mup.reshape(640, 4 * HSP)
    w_x = mup[:IN].astype(jnp.bfloat16)
    w_h = mup[IN:IN + PROJ].astype(jnp.bfloat16)
    b_row = mup[IN + PROJ:IN + PROJ + 1]

    pw = jnp.pad(proj_w.T, ((0, HSP - HS), (0, 0))).astype(jnp.bfloat16)
    pb = proj_b[None, :]

    xt = jnp.swapaxes(x, 0, 1)

    hid, h_fin, c_fin = pl.pallas_call(
        _lstm_body,
        grid=(nb, T),
        in_specs=[
            pl.BlockSpec((1, BB, IN), lambda b, t: (t, b, 0)),
            pl.BlockSpec((IN, 4 * HSP), lambda b, t: (0, 0)),
            pl.BlockSpec((PROJ, 4 * HSP), lambda b, t: (0, 0)),
            pl.BlockSpec((1, 4 * HSP), lambda b, t: (0, 0)),
            pl.BlockSpec((HSP, PROJ), lambda b, t: (0, 0)),
            pl.BlockSpec((1, PROJ), lambda b, t: (0, 0)),
        ],
        out_specs=[
            pl.BlockSpec((BB, 1, 1, PROJ), lambda b, t: (b, t, 0, 0)),
            pl.BlockSpec((BB, PROJ), lambda b, t: (b, 0)),
            pl.BlockSpec((BB, HS), lambda b, t: (b, 0)),
        ],
        out_shape=[
            jax.ShapeDtypeStruct((B, T, 1, PROJ), jnp.float32),
            jax.ShapeDtypeStruct((B, PROJ), jnp.float32),
            jax.ShapeDtypeStruct((B, HS), jnp.float32),
        ],
        scratch_shapes=[
            pltpu.VMEM((BB, PROJ), jnp.float32),
            pltpu.VMEM((BB, HSP), jnp.float32),
            pltpu.VMEM((BB, PROJ), jnp.float32),
        ],
        compiler_params=pltpu.CompilerParams(
            dimension_semantics=("parallel", "arbitrary"),
            vmem_limit_bytes=56 * 1024 * 1024,
        ),
        name="lstm_adc",
    )(xt, w_x, w_h, b_row, pw, pb)

    return hid.reshape(B, T, PROJ), h_fin, c_fin
